# Initial kernel scaffold; baseline (speedup 1.0000x reference)
import jax
import jax.numpy as jnp
from jax.experimental import pallas as pl


def kernel(boxes):
    raise NotImplementedError("write your pallas kernel here")


# speedup vs baseline: 4.4334x; 4.4334x over previous
import functools

import jax
import jax.numpy as jnp
from jax import lax
from jax.experimental import pallas as pl
from jax.experimental.pallas import tpu as pltpu
from jax.experimental.pallas import tpu_sc as plsc


def _encoding_table(length: int, size: int) -> jax.Array:
    depth = size // 2
    positions = jnp.arange(length, dtype=jnp.float32)[:, None]
    depths = jnp.arange(depth, dtype=jnp.float32)[None, :] / depth
    angle_rates = 1.0 / (10000.0 ** depths)
    angle_rads = positions * angle_rates
    return jnp.concatenate([jnp.sin(angle_rads), jnp.cos(angle_rads)], axis=-1)


@functools.partial(jax.jit, static_argnums=(2, 3))
def _sc_gather(table, idx, b, size):
    info = plsc.get_sparse_core_info()
    nc, ns = info.num_cores, info.num_subcores
    nw = nc * ns
    b_per_w = b // nw
    chunk = 512
    n_chunks = b_per_w // chunk

    mesh = plsc.VectorSubcoreMesh(core_axis_name="c", subcore_axis_name="s")

    @functools.partial(
        pl.kernel,
        mesh=mesh,
        out_type=jax.ShapeDtypeStruct((b, size), jnp.float32),
        compiler_params=pltpu.CompilerParams(use_tc_tiling_on_sc=False),
        scratch_types=[
            pltpu.VMEM((chunk,), jnp.int32),
            pltpu.VMEM((chunk, size), jnp.float32),
            pltpu.SemaphoreType.DMA,
        ],
    )
    def k(table_hbm, idx_hbm, out_hbm, idx_v, rows_v, sem):
        wid = lax.axis_index("s") * nc + lax.axis_index("c")
        base = wid * b_per_w

        def body(i, carry):
            off = base + i * chunk
            pltpu.sync_copy(idx_hbm.at[pl.ds(off, chunk)], idx_v)
            pltpu.async_copy(table_hbm.at[idx_v], rows_v, sem).wait()
            pltpu.sync_copy(rows_v, out_hbm.at[pl.ds(off, chunk)])
            return carry

        lax.fori_loop(0, n_chunks, body, 0)

    return k(table, idx)


def kernel(boxes):
    n, w, d = boxes.shape
    size = 256 // d
    table = _encoding_table(2048, size)
    idx = jnp.round(boxes).astype(jnp.int32).reshape(-1)
    out = _sc_gather(table, idx, idx.shape[0], size)
    return out.reshape(n, w, d * size)


# speedup vs baseline: 4.5657x; 1.0298x over previous
import functools

import jax
import jax.numpy as jnp
from jax import lax
from jax.experimental import pallas as pl
from jax.experimental.pallas import tpu as pltpu
from jax.experimental.pallas import tpu_sc as plsc


def _encoding_table(length: int, size: int) -> jax.Array:
    depth = size // 2
    positions = jnp.arange(length, dtype=jnp.float32)[:, None]
    depths = jnp.arange(depth, dtype=jnp.float32)[None, :] / depth
    angle_rates = 1.0 / (10000.0 ** depths)
    angle_rads = positions * angle_rates
    return jnp.concatenate([jnp.sin(angle_rads), jnp.cos(angle_rads)], axis=-1)


@functools.partial(jax.jit, static_argnums=(2, 3))
def _sc_gather(table, idx, b, size):
    info = plsc.get_sparse_core_info()
    nc, ns = info.num_cores, info.num_subcores
    nw = nc * ns
    b_per_w = b // nw
    chunk = 512
    n_chunks = b_per_w // chunk

    n_rounds = n_chunks // 2

    mesh = plsc.VectorSubcoreMesh(core_axis_name="c", subcore_axis_name="s")

    @functools.partial(
        pl.kernel,
        mesh=mesh,
        out_type=jax.ShapeDtypeStruct((b, size), jnp.float32),
        compiler_params=pltpu.CompilerParams(use_tc_tiling_on_sc=False),
        scratch_types=[
            pltpu.VMEM((chunk,), jnp.int32),
            pltpu.VMEM((chunk,), jnp.int32),
            pltpu.VMEM((chunk, size), jnp.float32),
            pltpu.VMEM((chunk, size), jnp.float32),
            pltpu.SemaphoreType.DMA,
            pltpu.SemaphoreType.DMA,
            pltpu.SemaphoreType.DMA,
            pltpu.SemaphoreType.DMA,
            pltpu.SemaphoreType.DMA,
            pltpu.SemaphoreType.DMA,
        ],
    )
    def k(table_hbm, idx_hbm, out_hbm, i0, i1, r0, r1,
          si0, si1, sg0, sg1, so0, so1):
        idx_v, rows_v = (i0, i1), (r0, r1)
        sidx, sgat, sout = (si0, si1), (sg0, sg1), (so0, so1)
        wid = lax.axis_index("s") * nc + lax.axis_index("c")
        base = wid * b_per_w

        def start_idx(slot, i):
            pltpu.async_copy(
                idx_hbm.at[pl.ds(base + i * chunk, chunk)], idx_v[slot],
                sidx[slot])

        def wait_idx(slot, i):
            pltpu.make_async_copy(
                idx_hbm.at[pl.ds(base + i * chunk, chunk)], idx_v[slot],
                sidx[slot]).wait()

        def start_out(slot, i):
            pltpu.async_copy(
                rows_v[slot], out_hbm.at[pl.ds(base + i * chunk, chunk)],
                sout[slot])

        def wait_out(slot, i):
            pltpu.make_async_copy(
                rows_v[slot], out_hbm.at[pl.ds(base + i * chunk, chunk)],
                sout[slot]).wait()

        def gather(slot):
            pltpu.async_copy(table_hbm.at[idx_v[slot]], rows_v[slot],
                             sgat[slot]).wait()

        start_idx(0, 0)
        start_idx(1, 1)
        for slot in (0, 1):
            wait_idx(slot, slot)
            gather(slot)
            start_idx(slot, slot + 2)
            start_out(slot, slot)

        def body(r, carry):
            for slot in (0, 1):
                i = 2 * r + slot
                wait_idx(slot, i)
                wait_out(slot, i - 2)
                gather(slot)
                start_idx(slot, i + 2)
                start_out(slot, i)
            return carry

        lax.fori_loop(1, n_rounds - 1, body, 0)

        for slot in (0, 1):
            i = 2 * (n_rounds - 1) + slot
            wait_idx(slot, i)
            wait_out(slot, i - 2)
            gather(slot)
            start_out(slot, i)
        for slot in (0, 1):
            wait_out(slot, 2 * (n_rounds - 1) + slot)

    return k(table, idx)


def kernel(boxes):
    n, w, d = boxes.shape
    size = 256 // d
    table = _encoding_table(2048, size)
    idx = jnp.round(boxes).astype(jnp.int32).reshape(-1)
    out = _sc_gather(table, idx, idx.shape[0], size)
    return out.reshape(n, w, d * size)


# speedup vs baseline: 5.3515x; 1.1721x over previous
import functools

import jax
import jax.numpy as jnp
from jax import lax
from jax.experimental import pallas as pl
from jax.experimental.pallas import tpu as pltpu
from jax.experimental.pallas import tpu_sc as plsc


def _encoding_table(length: int, size: int) -> jax.Array:
    depth = size // 2
    positions = jnp.arange(length, dtype=jnp.float32)[:, None]
    depths = jnp.arange(depth, dtype=jnp.float32)[None, :] / depth
    angle_rates = 1.0 / (10000.0 ** depths)
    angle_rads = positions * angle_rates
    return jnp.concatenate([jnp.sin(angle_rads), jnp.cos(angle_rads)], axis=-1)


@functools.partial(jax.jit, static_argnums=(2, 3))
def _sc_gather(table, idx, b, size):
    info = plsc.get_sparse_core_info()
    nc, ns = info.num_cores, info.num_subcores
    nw = nc * ns
    b_per_w = b // nw
    chunk = 512
    n_chunks = b_per_w // chunk

    n_rounds = n_chunks // 2

    mesh = plsc.VectorSubcoreMesh(core_axis_name="c", subcore_axis_name="s")

    @functools.partial(
        pl.kernel,
        mesh=mesh,
        out_type=jax.ShapeDtypeStruct((b, size), jnp.float32),
        compiler_params=pltpu.CompilerParams(use_tc_tiling_on_sc=False),
        scratch_types=[
            pltpu.VMEM((chunk,), jnp.int32),
            pltpu.VMEM((chunk,), jnp.int32),
            pltpu.VMEM((chunk, size), jnp.float32),
            pltpu.VMEM((chunk, size), jnp.float32),
            pltpu.SemaphoreType.DMA,
            pltpu.SemaphoreType.DMA,
            pltpu.SemaphoreType.DMA,
            pltpu.SemaphoreType.DMA,
            pltpu.SemaphoreType.DMA,
            pltpu.SemaphoreType.DMA,
        ],
    )
    def k(table_hbm, idx_hbm, out_hbm, i0, i1, r0, r1,
          si0, si1, sg0, sg1, so0, so1):
        idx_v, rows_v = (i0, i1), (r0, r1)
        sidx, sgat, sout = (si0, si1), (sg0, sg1), (so0, so1)
        wid = lax.axis_index("s") * nc + lax.axis_index("c")
        base = wid * b_per_w

        def start_idx(slot, i):
            pltpu.async_copy(
                idx_hbm.at[pl.ds(base + i * chunk, chunk)], idx_v[slot],
                sidx[slot])

        def wait_idx(slot, i):
            pltpu.make_async_copy(
                idx_hbm.at[pl.ds(base + i * chunk, chunk)], idx_v[slot],
                sidx[slot]).wait()

        def start_out(slot, i):
            pltpu.async_copy(
                rows_v[slot], out_hbm.at[pl.ds(base + i * chunk, chunk)],
                sout[slot])

        def wait_out(slot, i):
            pltpu.make_async_copy(
                rows_v[slot], out_hbm.at[pl.ds(base + i * chunk, chunk)],
                sout[slot]).wait()

        def gather(slot):
            pltpu.async_copy(table_hbm.at[idx_v[slot]], rows_v[slot],
                             sgat[slot]).wait()

        start_idx(0, 0)
        start_idx(1, 1)
        for slot in (0, 1):
            wait_idx(slot, slot)
            gather(slot)
            start_idx(slot, slot + 2)
            start_out(slot, slot)

        def body(r, carry):
            for slot in (0, 1):
                i = 2 * r + slot
                wait_idx(slot, i)
                wait_out(slot, i - 2)
                gather(slot)
                start_idx(slot, i + 2)
                start_out(slot, i)
            return carry

        lax.fori_loop(1, n_rounds - 1, body, 0)

        for slot in (0, 1):
            i = 2 * (n_rounds - 1) + slot
            wait_idx(slot, i)
            wait_out(slot, i - 2)
            gather(slot)
            start_out(slot, i)
        for slot in (0, 1):
            wait_out(slot, 2 * (n_rounds - 1) + slot)

    return k(table, idx)


def kernel(boxes):
    n, w, d = boxes.shape
    size = 256 // d
    table = _encoding_table(2048, size)
    idx = jnp.transpose(jnp.round(boxes).astype(jnp.int32), (1, 0, 2)).reshape(-1)
    out = _sc_gather(table, idx, idx.shape[0], size)
    return jnp.transpose(out.reshape(w, n, d * size), (1, 0, 2))
